# baseline (speedup 1.0000x reference)
import functools

import jax
import jax.numpy as jnp
from jax import lax
from jax.experimental import pallas as pl
from jax.experimental.pallas import tpu as pltpu, tpu_sc as plsc

NUM_EMBED = 8192
EMBED_DIM = 64

_info = plsc.get_sparse_core_info()
_NC, _NS = _info.num_cores, _info.num_subcores
_NW = _NC * _NS

_B = 256 * 32 * 32
_BPW = _B // _NW
_C = 64
_NCHUNK = _BPW // _C


def _gather_sc(table_padded, idx):
    mesh = plsc.VectorSubcoreMesh(core_axis_name="c", subcore_axis_name="s")

    @functools.partial(
        pl.kernel,
        mesh=mesh,
        out_type=jax.ShapeDtypeStruct((_B, 2 * EMBED_DIM), jnp.float32),
        scratch_types=[
            pltpu.VMEM((_BPW,), jnp.int32),
            pltpu.VMEM((_C, 2 * EMBED_DIM), jnp.float32),
            pltpu.VMEM((_C, 2 * EMBED_DIM), jnp.float32),
            pltpu.VMEM_SHARED((NUM_EMBED, 2 * EMBED_DIM), jnp.float32),
            pltpu.SemaphoreType.DMA,
            pltpu.SemaphoreType.DMA,
            pltpu.SemaphoreType.DMA,
            pltpu.SemaphoreType.DMA,
        ],
    )
    def k(table_hbm, idx_hbm, out_hbm, idx_all, rows0, rows1, spt, sg0, sg1,
          ss0, ss1):
        sid = lax.axis_index("s")
        wid = sid * _NC + lax.axis_index("c")
        base = wid * _BPW

        rpw = NUM_EMBED // _NS
        pltpu.sync_copy(table_hbm.at[pl.ds(sid * rpw, rpw)],
                        spt.at[pl.ds(sid * rpw, rpw)])
        pltpu.sync_copy(idx_hbm.at[pl.ds(base, _BPW)], idx_all)
        plsc.subcore_barrier()

        rows, sg, ss = [rows0, rows1], [sg0, sg1], [ss0, ss1]
        gathers = [None] * _NCHUNK
        stores = [None] * _NCHUNK
        for i in range(_NCHUNK):
            b = i % 2
            if i >= 2:
                stores[i - 2].wait()
            gathers[i] = pltpu.async_copy(
                spt.at[idx_all.at[pl.ds(i * _C, _C)]], rows[b], sg[b])
            if i >= 1:
                gathers[i - 1].wait()
                stores[i - 1] = pltpu.async_copy(
                    rows[1 - b],
                    out_hbm.at[pl.ds(base + (i - 1) * _C, _C)],
                    ss[1 - b])
        last = _NCHUNK - 1
        gathers[last].wait()
        stores[last] = pltpu.async_copy(
            rows[last % 2],
            out_hbm.at[pl.ds(base + last * _C, _C)],
            ss[last % 2])
        stores[last - 1].wait()
        stores[last].wait()

    return k(table_padded, idx)


def kernel(templat, input):
    idx = input.reshape(-1).astype(jnp.int32)
    table_padded = jnp.pad(templat, ((0, 0), (0, EMBED_DIM)))
    out = _gather_sc(table_padded, idx)
    return out[:, :EMBED_DIM].reshape(input.shape + (EMBED_DIM,))


# speedup vs baseline: 1.0233x; 1.0233x over previous
import functools

import jax
import jax.numpy as jnp
from jax import lax
from jax.experimental import pallas as pl
from jax.experimental.pallas import tpu as pltpu, tpu_sc as plsc

NUM_EMBED = 8192
EMBED_DIM = 64

_info = plsc.get_sparse_core_info()
_NC, _NS = _info.num_cores, _info.num_subcores
_NW = _NC * _NS

_B = 256 * 32 * 32
_BPW = _B // _NW
_C = 128
_NCHUNK = _BPW // _C


def _gather_sc(table_padded, idx):
    mesh = plsc.VectorSubcoreMesh(core_axis_name="c", subcore_axis_name="s")

    @functools.partial(
        pl.kernel,
        mesh=mesh,
        out_type=jax.ShapeDtypeStruct((_B, 2 * EMBED_DIM), jnp.float32),
        scratch_types=[
            pltpu.VMEM((_BPW,), jnp.int32),
            pltpu.VMEM((_C, 2 * EMBED_DIM), jnp.float32),
            pltpu.VMEM((_C, 2 * EMBED_DIM), jnp.float32),
            pltpu.VMEM_SHARED((NUM_EMBED, 2 * EMBED_DIM), jnp.float32),
            pltpu.SemaphoreType.DMA,
            pltpu.SemaphoreType.DMA,
            pltpu.SemaphoreType.DMA,
            pltpu.SemaphoreType.DMA,
        ],
    )
    def k(table_hbm, idx_hbm, out_hbm, idx_all, rows0, rows1, spt, sg0, sg1,
          ss0, ss1):
        sid = lax.axis_index("s")
        wid = sid * _NC + lax.axis_index("c")
        base = wid * _BPW

        rpw = NUM_EMBED // _NS
        pltpu.sync_copy(table_hbm.at[pl.ds(sid * rpw, rpw)],
                        spt.at[pl.ds(sid * rpw, rpw)])
        pltpu.sync_copy(idx_hbm.at[pl.ds(base, _BPW)], idx_all)
        plsc.subcore_barrier()

        rows, sg, ss = [rows0, rows1], [sg0, sg1], [ss0, ss1]
        gathers = [None] * _NCHUNK
        stores = [None] * _NCHUNK
        for i in range(_NCHUNK):
            b = i % 2
            if i >= 2:
                stores[i - 2].wait()
            gathers[i] = pltpu.async_copy(
                spt.at[idx_all.at[pl.ds(i * _C, _C)]], rows[b], sg[b])
            if i >= 1:
                gathers[i - 1].wait()
                stores[i - 1] = pltpu.async_copy(
                    rows[1 - b],
                    out_hbm.at[pl.ds(base + (i - 1) * _C, _C)],
                    ss[1 - b])
        last = _NCHUNK - 1
        gathers[last].wait()
        stores[last] = pltpu.async_copy(
            rows[last % 2],
            out_hbm.at[pl.ds(base + last * _C, _C)],
            ss[last % 2])
        stores[last - 1].wait()
        stores[last].wait()

    return k(table_padded, idx)


def kernel(templat, input):
    idx = input.reshape(-1).astype(jnp.int32)
    table_padded = jnp.pad(templat, ((0, 0), (0, EMBED_DIM)))
    out = _gather_sc(table_padded, idx)
    return out[:, :EMBED_DIM].reshape(input.shape + (EMBED_DIM,))
